# Initial kernel scaffold; baseline (speedup 1.0000x reference)
import jax
import jax.numpy as jnp
from jax.experimental import pallas as pl


def kernel(train, month_w, day_w, hour_w, type_w):
    raise NotImplementedError("write your pallas kernel here")


# speedup vs baseline: 2.8005x; 2.8005x over previous
import functools

import jax
import jax.numpy as jnp
from jax import lax
from jax.experimental import pallas as pl
from jax.experimental.pallas import tpu as pltpu
from jax.experimental.pallas import tpu_sc as plsc

_L = 16


def _make_sc_gather(num_tokens: int, table_rows: int, dim: int, feat: int):
    info = plsc.get_sparse_core_info()
    nc, ns = info.num_cores, info.num_subcores
    nw = nc * ns
    per_w = num_tokens // nw
    assert per_w * nw == num_tokens

    chunk = 2560
    while per_w % chunk != 0:
        chunk //= 2
    n_chunks = per_w // chunk
    n_groups = chunk // _L

    mesh = plsc.VectorSubcoreMesh(core_axis_name="c", subcore_axis_name="s")

    @functools.partial(
        pl.kernel,
        mesh=mesh,
        out_type=jax.ShapeDtypeStruct((num_tokens * dim,), jnp.float32),
        scratch_types=[
            pltpu.VMEM((chunk * feat,), jnp.float32),
            pltpu.VMEM((table_rows * dim,), jnp.float32),
            pltpu.VMEM((chunk * dim,), jnp.float32),
        ],
        compiler_params=pltpu.CompilerParams(needs_layout_passes=False),
    )
    def sc_kernel(train_hbm, table_hbm, out_hbm, train_v, table_v, out_v):
        wid = lax.axis_index("s") * nc + lax.axis_index("c")
        pltpu.sync_copy(table_hbm, table_v)

        iota = lax.iota(jnp.int32, _L)
        tpos = iota * feat + 6
        opos = iota * dim

        def group_body(g, _):
            fidx = plsc.load_gather(train_v, [tpos + g * (_L * feat)])
            rows = fidx.astype(jnp.int32) * dim
            obase = opos + g * (_L * dim)
            for d in range(dim):
                v = plsc.load_gather(table_v, [rows + d])
                plsc.store_scatter(out_v, [obase + d], v)
            return 0

        def chunk_body(ci, _):
            base = wid * per_w + ci * chunk
            pltpu.sync_copy(train_hbm.at[pl.ds(base * feat, chunk * feat)], train_v)
            lax.fori_loop(0, n_groups, group_body, 0, unroll=4)
            pltpu.sync_copy(out_v, out_hbm.at[pl.ds(base * dim, chunk * dim)])
            return 0

        lax.fori_loop(0, n_chunks, chunk_body, 0)

    return sc_kernel


def kernel(train, month_w, day_w, hour_w, type_w):
    b, t, f = train.shape
    rows, dim = type_w.shape
    flat = train.reshape(b * t * f)
    sc = _make_sc_gather(b * t, rows, dim, f)
    out = sc(flat, type_w.reshape(rows * dim))
    return out.reshape(b, t, dim)


# speedup vs baseline: 30.6107x; 10.9306x over previous
import functools

import jax
import jax.numpy as jnp
from jax import lax
from jax.experimental import pallas as pl
from jax.experimental.pallas import tpu as pltpu
from jax.experimental.pallas import tpu_sc as plsc

_L = 16


def _make_sc_gather(n_tiles: int, table_rows: int, dim: int):
    info = plsc.get_sparse_core_info()
    nc, ns = info.num_cores, info.num_subcores
    nw = nc * ns
    per_w = n_tiles // nw
    assert per_w * nw == n_tiles

    kc = 25
    while per_w % kc != 0:
        kc -= 1
    n_chunks = per_w // kc

    mesh = plsc.VectorSubcoreMesh(core_axis_name="c", subcore_axis_name="s")

    @functools.partial(
        pl.kernel,
        mesh=mesh,
        out_type=jax.ShapeDtypeStruct((n_tiles * dim * 128,), jnp.float32),
        scratch_types=[
            pltpu.VMEM((kc * 128,), jnp.float32),
            pltpu.VMEM((dim * table_rows,), jnp.float32),
            pltpu.VMEM((kc * dim * 128,), jnp.float32),
        ],
        compiler_params=pltpu.CompilerParams(needs_layout_passes=False),
    )
    def sc_kernel(idx_hbm, table_hbm, out_hbm, idx_v, table_v, out_v):
        wid = lax.axis_index("s") * nc + lax.axis_index("c")
        pltpu.sync_copy(table_hbm, table_v)

        def tile_body(k, _):
            for c in range(128 // _L):
                f = idx_v[pl.ds(k * 128 + c * _L, _L)]
                rows = f.astype(jnp.int32)
                for d in range(dim):
                    v = plsc.load_gather(table_v, [rows + d * table_rows])
                    out_v[pl.ds(k * dim * 128 + d * 128 + c * _L, _L)] = v
            return 0

        def chunk_body(ci, _):
            base_k = wid * per_w + ci * kc
            pltpu.sync_copy(idx_hbm.at[pl.ds(base_k * 128, kc * 128)], idx_v)
            lax.fori_loop(0, kc, tile_body, 0, unroll=2)
            pltpu.sync_copy(out_v, out_hbm.at[pl.ds(base_k * dim * 128, kc * dim * 128)])
            return 0

        lax.fori_loop(0, n_chunks, chunk_body, 0)

    return sc_kernel


def kernel(train, month_w, day_w, hour_w, type_w):
    b, t, f = train.shape
    rows, dim = type_w.shape
    bg = b // 128
    idx_lin = train[:, :, 6].T.reshape(t * b)
    table_lin = type_w.T.reshape(dim * rows)
    sc = _make_sc_gather(t * bg, rows, dim)
    out_flat = sc(idx_lin, table_lin)
    return out_flat.reshape(t, bg, dim, 128).transpose(1, 3, 0, 2).reshape(b, t, dim)


# speedup vs baseline: 32.9057x; 1.0750x over previous
import functools

import jax
import jax.numpy as jnp
from jax import lax
from jax.experimental import pallas as pl
from jax.experimental.pallas import tpu as pltpu
from jax.experimental.pallas import tpu_sc as plsc

_L = 16


def _make_sc_gather(n_tiles: int, table_rows: int, dim: int):
    info = plsc.get_sparse_core_info()
    nc, ns = info.num_cores, info.num_subcores
    nw = nc * ns
    per_w = n_tiles // nw
    assert per_w * nw == n_tiles

    kc = 25
    while per_w % kc != 0:
        kc -= 1
    n_chunks = per_w // kc

    mesh = plsc.VectorSubcoreMesh(core_axis_name="c", subcore_axis_name="s")

    @functools.partial(
        pl.kernel,
        mesh=mesh,
        out_type=jax.ShapeDtypeStruct((n_tiles * dim * 128,), jnp.float32),
        scratch_types=[
            pltpu.VMEM((kc * 128,), jnp.float32),
            pltpu.VMEM((kc * 128,), jnp.float32),
            pltpu.VMEM((kc * dim * 128,), jnp.float32),
            pltpu.VMEM((kc * dim * 128,), jnp.float32),
            pltpu.VMEM((dim * table_rows,), jnp.float32),
            pltpu.SemaphoreType.DMA,
            pltpu.SemaphoreType.DMA,
            pltpu.SemaphoreType.DMA,
            pltpu.SemaphoreType.DMA,
        ],
        compiler_params=pltpu.CompilerParams(needs_layout_passes=False),
    )
    def sc_kernel(idx_hbm, table_hbm, out_hbm, idx_v0, idx_v1, out_v0,
                  out_v1, table_v, sem_i0, sem_i1, sem_o0, sem_o1):
        wid = lax.axis_index("s") * nc + lax.axis_index("c")
        pltpu.sync_copy(table_hbm, table_v)
        idx_bufs, out_bufs = [idx_v0, idx_v1], [out_v0, out_v1]
        isems, osems = [sem_i0, sem_i1], [sem_o0, sem_o1]
        base = wid * per_w

        def start_in(ci, buf):
            return pltpu.async_copy(
                idx_hbm.at[pl.ds((base + ci * kc) * 128, kc * 128)],
                idx_bufs[buf], isems[buf])

        def start_out(ci, buf):
            return pltpu.async_copy(
                out_bufs[buf],
                out_hbm.at[pl.ds((base + ci * kc) * dim * 128, kc * dim * 128)],
                osems[buf])

        def compute(idx_v, out_v):
            def tile_body(k, _):
                for c in range(128 // _L):
                    f = idx_v[pl.ds(k * 128 + c * _L, _L)]
                    rows = f.astype(jnp.int32)
                    for d in range(dim):
                        v = plsc.load_gather(table_v, [rows + d * table_rows])
                        out_v[pl.ds(k * dim * 128 + d * 128 + c * _L, _L)] = v
                return 0
            lax.fori_loop(0, kc, tile_body, 0, unroll=2)

        h_in = [start_in(0, 0), None]
        h_out = [None, None]
        for ci in range(n_chunks):
            b = ci & 1
            h_in[b].wait()
            if ci + 1 < n_chunks:
                h_in[1 - b] = start_in(ci + 1, 1 - b)
            if h_out[b] is not None:
                h_out[b].wait()
            compute(idx_bufs[b], out_bufs[b])
            h_out[b] = start_out(ci, b)
        h_out[(n_chunks - 1) & 1].wait()
        if n_chunks > 1:
            h_out[n_chunks & 1].wait()

    return sc_kernel


def kernel(train, month_w, day_w, hour_w, type_w):
    b, t, f = train.shape
    rows, dim = type_w.shape
    bg = b // 128
    idx_lin = train[:, :, 6].T.reshape(t * b)
    table_lin = type_w.T.reshape(dim * rows)
    sc = _make_sc_gather(t * bg, rows, dim)
    out_flat = sc(idx_lin, table_lin)
    return out_flat.reshape(t, bg, dim, 128).transpose(1, 3, 0, 2).reshape(b, t, dim)


# speedup vs baseline: 50.7727x; 1.5430x over previous
import functools

import jax
import jax.numpy as jnp
from jax import lax
from jax.experimental import pallas as pl
from jax.experimental.pallas import tpu as pltpu
from jax.experimental.pallas import tpu_sc as plsc

_L = 16


def _make_sc_gather(n_tiles: int, table_rows: int, dim: int):
    info = plsc.get_sparse_core_info()
    nc, ns = info.num_cores, info.num_subcores
    nw = nc * ns
    per_w = n_tiles // nw
    assert per_w * nw == n_tiles

    kc = 25
    while per_w % kc != 0:
        kc -= 1
    n_chunks = per_w // kc

    mesh = plsc.VectorSubcoreMesh(core_axis_name="c", subcore_axis_name="s")

    @functools.partial(
        pl.kernel,
        mesh=mesh,
        out_type=jax.ShapeDtypeStruct((n_tiles * dim * 128,), jnp.float32),
        scratch_types=[
            pltpu.VMEM((kc * 128,), jnp.float32),
            pltpu.VMEM((kc * 128,), jnp.float32),
            pltpu.VMEM((kc * dim * 128,), jnp.float32),
            pltpu.VMEM((kc * dim * 128,), jnp.float32),
            pltpu.VMEM((dim * table_rows,), jnp.float32),
            pltpu.SemaphoreType.DMA,
            pltpu.SemaphoreType.DMA,
            pltpu.SemaphoreType.DMA,
            pltpu.SemaphoreType.DMA,
        ],
        compiler_params=pltpu.CompilerParams(needs_layout_passes=False),
    )
    def sc_kernel(idx_hbm, table_hbm, out_hbm, idx_v0, idx_v1, out_v0,
                  out_v1, table_v, sem_i0, sem_i1, sem_o0, sem_o1):
        wid = lax.axis_index("s") * nc + lax.axis_index("c")
        pltpu.sync_copy(table_hbm, table_v)
        idx_bufs, out_bufs = [idx_v0, idx_v1], [out_v0, out_v1]
        isems, osems = [sem_i0, sem_i1], [sem_o0, sem_o1]
        base = wid * per_w

        def start_in(ci, buf):
            return pltpu.async_copy(
                idx_hbm.at[pl.ds((base + ci * kc) * 128, kc * 128)],
                idx_bufs[buf], isems[buf])

        def start_out(ci, buf):
            return pltpu.async_copy(
                out_bufs[buf],
                out_hbm.at[pl.ds((base + ci * kc) * dim * 128, kc * dim * 128)],
                osems[buf])

        def compute(idx_v, out_v):
            def tile_body(k, _):
                for c in range(128 // _L):
                    f = idx_v[pl.ds(k * 128 + c * _L, _L)]
                    rows = f.astype(jnp.int32)
                    vals = [plsc.load_gather(table_v, [rows + d * table_rows])
                            for d in range(dim)]
                    for d in range(dim):
                        out_v[pl.ds(k * dim * 128 + d * 128 + c * _L, _L)] = vals[d]
                return 0
            lax.fori_loop(0, kc, tile_body, 0, unroll=2)

        h_in = [start_in(0, 0), None]
        h_out = [None, None]
        for ci in range(n_chunks):
            b = ci & 1
            h_in[b].wait()
            if ci + 1 < n_chunks:
                h_in[1 - b] = start_in(ci + 1, 1 - b)
            if h_out[b] is not None:
                h_out[b].wait()
            compute(idx_bufs[b], out_bufs[b])
            h_out[b] = start_out(ci, b)
        h_out[(n_chunks - 1) & 1].wait()
        if n_chunks > 1:
            h_out[n_chunks & 1].wait()

    return sc_kernel


def kernel(train, month_w, day_w, hour_w, type_w):
    b, t, f = train.shape
    rows, dim = type_w.shape
    bg = b // 128
    idx_lin = train[:, :, 6].T.reshape(t * b)
    table_lin = type_w.T.reshape(dim * rows)
    sc = _make_sc_gather(t * bg, rows, dim)
    out_flat = sc(idx_lin, table_lin)
    return out_flat.reshape(t, bg, dim, 128).transpose(1, 3, 0, 2).reshape(b, t, dim)
